# baseline (speedup 1.0000x reference)
import functools

import jax
import jax.numpy as jnp
from jax import lax
from jax.experimental import pallas as pl
from jax.experimental.pallas import tpu as pltpu
from jax.experimental.pallas import tpu_sc as plsc

N = 10000
E = 320000
D = 128
NC = 2
NS = 16
NW = NC * NS
EPT = E // NW
CHN = 128
NQ = 10
NCHP = NQ * 8
EPTP = NCHP * CHN
NPAD = 10112
RPT = NPAD // NS

_mesh = plsc.VectorSubcoreMesh(
    core_axis_name="c", subcore_axis_name="s", num_cores=NC, num_subcores=NS
)


@functools.partial(
    pl.kernel,
    out_type=jax.ShapeDtypeStruct((NW, N), jnp.float32),
    mesh=_mesh,
    scratch_types=[
        pltpu.VMEM((EPT,), jnp.int32),
        pltpu.VMEM((N,), jnp.float32),
    ],
    compiler_params=pltpu.CompilerParams(needs_layout_passes=False),
)
def _deg_kernel(dst_hbm, zn_hbm, deg_hbm, dstv, degv):
    cid = lax.axis_index("c")
    sid = lax.axis_index("s")
    w = sid * NC + cid
    pltpu.sync_copy(dst_hbm.at[w], dstv)
    pltpu.sync_copy(zn_hbm, degv)
    ones = jnp.ones((16,), jnp.float32)

    @pl.loop(0, EPT // 16)
    def _(i):
        idx = dstv[pl.ds(i * 16, 16)]
        plsc.addupdate_scatter(degv, [idx], ones)

    pltpu.sync_copy(degv, deg_hbm.at[w])


@functools.partial(
    pl.kernel,
    out_type=[
        jax.ShapeDtypeStruct((NPAD, D), jnp.float32),
        jax.ShapeDtypeStruct((NPAD, D), jnp.float32),
    ],
    mesh=_mesh,
    scratch_types=[
        pltpu.VMEM((NCHP, CHN), jnp.int32),
        pltpu.VMEM((NCHP, CHN), jnp.int32),
        pltpu.VMEM((CHN, D), jnp.float32),
        pltpu.VMEM_SHARED((NPAD, D), jnp.float32),
        [pltpu.SemaphoreType.DMA] * 2,
        [pltpu.SemaphoreType.DMA] * 2,
    ],
    compiler_params=pltpu.CompilerParams(needs_layout_passes=False),
)
def _edge_kernel(src_hbm, dst_hbm, p_hbm, zr_hbm, acc0_hbm, acc1_hbm,
                 srcv, dstv, rows, acc_sh, isems, gsems):
    cid = lax.axis_index("c")
    sid = lax.axis_index("s")
    w = sid * NC + cid
    pltpu.sync_copy(src_hbm.at[w], srcv)
    pltpu.sync_copy(dst_hbm.at[w], dstv)
    pltpu.sync_copy(zr_hbm, acc_sh.at[pl.ds(sid * RPT, RPT)])
    plsc.subcore_barrier()

    @pl.loop(0, NCHP)
    def _(c):
        pltpu.async_copy(p_hbm.at[srcv.at[c]], rows, gsems[0]).wait()
        pltpu.sync_copy(rows, acc_sh.at[dstv.at[c]], add=True)

    plsc.subcore_barrier()

    @pl.when(cid == 0)
    def _():
        pltpu.sync_copy(acc_sh.at[pl.ds(sid * RPT, RPT)],
                        acc0_hbm.at[pl.ds(sid * RPT, RPT)])

    @pl.when(cid == 1)
    def _():
        pltpu.sync_copy(acc_sh.at[pl.ds(sid * RPT, RPT)],
                        acc1_hbm.at[pl.ds(sid * RPT, RPT)])


BR = 1024
GRID = (N + BR - 1) // BR


def _row_mask():
    i = pl.program_id(0)
    rid = lax.broadcasted_iota(jnp.int32, (BR, 1), 0) + i * BR
    return rid < N


def _tc_first_body(degp_ref, x_ref, w_ref, p_ref, dinv_ref):
    deg = jnp.sum(degp_ref[...], axis=0) + 1.0
    dinv = lax.rsqrt(deg)[:, None]
    h = jnp.dot(x_ref[...], w_ref[...], preferred_element_type=jnp.float32)
    p_ref[...] = jnp.where(_row_mask(), h * dinv, 0.0)
    dinv_ref[...] = dinv


_tc_first = pl.pallas_call(
    _tc_first_body,
    grid=(GRID,),
    in_specs=[
        pl.BlockSpec((NW, BR), lambda i: (0, i)),
        pl.BlockSpec((BR, D), lambda i: (i, 0)),
        pl.BlockSpec((D, D), lambda i: (0, 0)),
    ],
    out_specs=[
        pl.BlockSpec((BR, D), lambda i: (i, 0)),
        pl.BlockSpec((BR, 1), lambda i: (i, 0)),
    ],
    out_shape=[
        jax.ShapeDtypeStruct((NPAD, D), jnp.float32),
        jax.ShapeDtypeStruct((N, 1), jnp.float32),
    ],
)


def _tc_mid_body(a0_ref, a1_ref, p_ref, dinv_ref, b_ref, w_ref, out_ref):
    dinv = dinv_ref[...]
    z = dinv * (a0_ref[...] + a1_ref[...] + p_ref[...]) + b_ref[...]
    h = jnp.maximum(z, 0.0)
    out_ref[...] = jnp.where(
        _row_mask(),
        jnp.dot(h, w_ref[...], preferred_element_type=jnp.float32) * dinv,
        0.0)


_tc_mid = pl.pallas_call(
    _tc_mid_body,
    grid=(GRID,),
    in_specs=[
        pl.BlockSpec((BR, D), lambda i: (i, 0)),
        pl.BlockSpec((BR, D), lambda i: (i, 0)),
        pl.BlockSpec((BR, D), lambda i: (i, 0)),
        pl.BlockSpec((BR, 1), lambda i: (i, 0)),
        pl.BlockSpec((1, D), lambda i: (0, 0)),
        pl.BlockSpec((D, D), lambda i: (0, 0)),
    ],
    out_specs=pl.BlockSpec((BR, D), lambda i: (i, 0)),
    out_shape=jax.ShapeDtypeStruct((NPAD, D), jnp.float32),
)


def _tc_last_body(a0_ref, a1_ref, p_ref, dinv_ref, b_ref, out_ref):
    z = dinv_ref[...] * (a0_ref[...] + a1_ref[...] + p_ref[...]) + b_ref[...]
    out_ref[...] = z


_tc_last = pl.pallas_call(
    _tc_last_body,
    grid=(GRID,),
    in_specs=[
        pl.BlockSpec((BR, D), lambda i: (i, 0)),
        pl.BlockSpec((BR, D), lambda i: (i, 0)),
        pl.BlockSpec((BR, D), lambda i: (i, 0)),
        pl.BlockSpec((BR, 1), lambda i: (i, 0)),
        pl.BlockSpec((1, D), lambda i: (0, 0)),
    ],
    out_specs=pl.BlockSpec((BR, D), lambda i: (i, 0)),
    out_shape=jax.ShapeDtypeStruct((N, D), jnp.float32),
)


def kernel(x, edge_index, W1, b1, W2, b2):
    src2 = edge_index[0].reshape(NW, EPT)
    dst2 = edge_index[1].reshape(NW, EPT)
    pad_s = jnp.full((NW, EPTP - EPT), N, jnp.int32)
    pad_d = jnp.full((NW, EPTP - EPT), NPAD - 1, jnp.int32)
    src4 = jnp.concatenate([src2, pad_s], axis=1).reshape(NW, NCHP, CHN)
    dst3 = jnp.concatenate([dst2, pad_d], axis=1).reshape(NW, NCHP, CHN)
    zn = jnp.zeros((N,), jnp.float32)
    zr = jnp.zeros((RPT, D), jnp.float32)
    b1r = b1.reshape(1, D)
    b2r = b2.reshape(1, D)

    degp = _deg_kernel(dst2, zn)
    p1, dinv = _tc_first(degp, x, W1)
    a0, a1 = _edge_kernel(src4, dst3, p1, zr)
    p2 = _tc_mid(a0, a1, p1, dinv, b1r, W2)
    c0, c1 = _edge_kernel(src4, dst3, p2, zr)
    out = _tc_last(c0, c1, p2, dinv, b2r)
    return out


# speedup vs baseline: 1.1162x; 1.1162x over previous
import functools

import jax
import jax.numpy as jnp
from jax import lax
from jax.experimental import pallas as pl
from jax.experimental.pallas import tpu as pltpu
from jax.experimental.pallas import tpu_sc as plsc

N = 10000
E = 320000
D = 128
NC = 2
NS = 16
NW = NC * NS
EPT = E // NW
CH = 80
NQ = 16
NCHP = NQ * 8
EPTP = NCHP * CH
NPAD = 10112
RPT = NPAD // NS

_mesh = plsc.VectorSubcoreMesh(
    core_axis_name="c", subcore_axis_name="s", num_cores=NC, num_subcores=NS
)


@functools.partial(
    pl.kernel,
    out_type=jax.ShapeDtypeStruct((NW, N), jnp.float32),
    mesh=_mesh,
    scratch_types=[
        pltpu.VMEM((EPT,), jnp.int32),
        pltpu.VMEM((N,), jnp.float32),
    ],
    compiler_params=pltpu.CompilerParams(needs_layout_passes=False),
)
def _deg_kernel(dst_hbm, zn_hbm, deg_hbm, dstv, degv):
    cid = lax.axis_index("c")
    sid = lax.axis_index("s")
    w = sid * NC + cid
    pltpu.sync_copy(dst_hbm.at[w], dstv)
    pltpu.sync_copy(zn_hbm, degv)
    ones = jnp.ones((16,), jnp.float32)

    @pl.loop(0, EPT // 16)
    def _(i):
        idx = dstv[pl.ds(i * 16, 16)]
        plsc.addupdate_scatter(degv, [idx], ones)

    pltpu.sync_copy(degv, deg_hbm.at[w])


@functools.partial(
    pl.kernel,
    out_type=[
        jax.ShapeDtypeStruct((NPAD, D), jnp.float32),
        jax.ShapeDtypeStruct((NPAD, D), jnp.float32),
    ],
    mesh=_mesh,
    scratch_types=[
        pltpu.VMEM((2, 8, CH), jnp.int32),
        pltpu.VMEM((NCHP, CH), jnp.int32),
        pltpu.VMEM((2 * CH, D), jnp.float32),
        pltpu.VMEM_SHARED((NPAD, D), jnp.float32),
        [pltpu.SemaphoreType.DMA] * 2,
        [pltpu.SemaphoreType.DMA] * 2,
    ],
    compiler_params=pltpu.CompilerParams(needs_layout_passes=False),
)
def _edge_kernel(src_hbm, dst_hbm, p_hbm, zr_hbm, acc0_hbm, acc1_hbm,
                 srcv, dstv, rows, acc_sh, isems, gsems):
    cid = lax.axis_index("c")
    sid = lax.axis_index("s")
    w = sid * NC + cid
    pltpu.sync_copy(dst_hbm.at[w], dstv)
    pltpu.sync_copy(zr_hbm, acc_sh.at[pl.ds(sid * RPT, RPT)])
    plsc.subcore_barrier()

    bufs = [rows.at[pl.ds(b * CH, CH)] for b in range(2)]

    def idx_start(q, s):
        pltpu.async_copy(src_hbm.at[w, q], srcv.at[s], isems[s])

    def idx_wait(q, s):
        pltpu.make_async_copy(src_hbm.at[w, q], srcv.at[s], isems[s]).wait()

    def g_start(s, j):
        pltpu.async_copy(p_hbm.at[srcv.at[s, j]], bufs[j % 2], gsems[j % 2])

    def g_wait(s, j):
        pltpu.make_async_copy(p_hbm.at[srcv.at[s, j]], bufs[j % 2],
                              gsems[j % 2]).wait()

    def scat(c, j):
        pltpu.sync_copy(bufs[j % 2], acc_sh.at[dstv.at[c]], add=True)

    idx_start(0, 0)
    idx_wait(0, 0)
    idx_start(1, 1)
    g_start(0, 0)

    def super_chunk(qbase, s, pf_wait, start_next):
        for j in range(8):
            c = qbase * 8 + j
            if j == 7:
                if pf_wait is not None:
                    idx_wait(pf_wait, s ^ 1)
                    g_start(s ^ 1, 0)
            else:
                g_start(s, j + 1)
            g_wait(s, j)
            scat(c, j)
            if j == 7 and start_next:
                idx_start(pf_wait + 1, s)

    @pl.loop(0, NQ // 2 - 1)
    def _(g):
        q = g * 2
        super_chunk(q, 0, q + 1, True)
        super_chunk(q + 1, 1, q + 2, True)

    q = NQ - 2
    super_chunk(q, 0, q + 1, False)
    super_chunk(q + 1, 1, None, False)

    plsc.subcore_barrier()

    @pl.when(cid == 0)
    def _():
        pltpu.sync_copy(acc_sh.at[pl.ds(sid * RPT, RPT)],
                        acc0_hbm.at[pl.ds(sid * RPT, RPT)])

    @pl.when(cid == 1)
    def _():
        pltpu.sync_copy(acc_sh.at[pl.ds(sid * RPT, RPT)],
                        acc1_hbm.at[pl.ds(sid * RPT, RPT)])


BR = 1024
GRID = (N + BR - 1) // BR


def _row_mask():
    i = pl.program_id(0)
    rid = lax.broadcasted_iota(jnp.int32, (BR, 1), 0) + i * BR
    return rid < N


def _tc_first_body(degp_ref, x_ref, w_ref, p_ref, dinv_ref):
    deg = jnp.sum(degp_ref[...], axis=0) + 1.0
    dinv = lax.rsqrt(deg)[:, None]
    h = jnp.dot(x_ref[...], w_ref[...], preferred_element_type=jnp.float32)
    p_ref[...] = jnp.where(_row_mask(), h * dinv, 0.0)
    dinv_ref[...] = dinv


_tc_first = pl.pallas_call(
    _tc_first_body,
    grid=(GRID,),
    in_specs=[
        pl.BlockSpec((NW, BR), lambda i: (0, i)),
        pl.BlockSpec((BR, D), lambda i: (i, 0)),
        pl.BlockSpec((D, D), lambda i: (0, 0)),
    ],
    out_specs=[
        pl.BlockSpec((BR, D), lambda i: (i, 0)),
        pl.BlockSpec((BR, 1), lambda i: (i, 0)),
    ],
    out_shape=[
        jax.ShapeDtypeStruct((NPAD, D), jnp.float32),
        jax.ShapeDtypeStruct((N, 1), jnp.float32),
    ],
)


def _tc_mid_body(a0_ref, a1_ref, p_ref, dinv_ref, b_ref, w_ref, out_ref):
    dinv = dinv_ref[...]
    z = dinv * (a0_ref[...] + a1_ref[...] + p_ref[...]) + b_ref[...]
    h = jnp.maximum(z, 0.0)
    out_ref[...] = jnp.where(
        _row_mask(),
        jnp.dot(h, w_ref[...], preferred_element_type=jnp.float32) * dinv,
        0.0)


_tc_mid = pl.pallas_call(
    _tc_mid_body,
    grid=(GRID,),
    in_specs=[
        pl.BlockSpec((BR, D), lambda i: (i, 0)),
        pl.BlockSpec((BR, D), lambda i: (i, 0)),
        pl.BlockSpec((BR, D), lambda i: (i, 0)),
        pl.BlockSpec((BR, 1), lambda i: (i, 0)),
        pl.BlockSpec((1, D), lambda i: (0, 0)),
        pl.BlockSpec((D, D), lambda i: (0, 0)),
    ],
    out_specs=pl.BlockSpec((BR, D), lambda i: (i, 0)),
    out_shape=jax.ShapeDtypeStruct((NPAD, D), jnp.float32),
)


def _tc_last_body(a0_ref, a1_ref, p_ref, dinv_ref, b_ref, out_ref):
    z = dinv_ref[...] * (a0_ref[...] + a1_ref[...] + p_ref[...]) + b_ref[...]
    out_ref[...] = z


_tc_last = pl.pallas_call(
    _tc_last_body,
    grid=(GRID,),
    in_specs=[
        pl.BlockSpec((BR, D), lambda i: (i, 0)),
        pl.BlockSpec((BR, D), lambda i: (i, 0)),
        pl.BlockSpec((BR, D), lambda i: (i, 0)),
        pl.BlockSpec((BR, 1), lambda i: (i, 0)),
        pl.BlockSpec((1, D), lambda i: (0, 0)),
    ],
    out_specs=pl.BlockSpec((BR, D), lambda i: (i, 0)),
    out_shape=jax.ShapeDtypeStruct((N, D), jnp.float32),
)


def kernel(x, edge_index, W1, b1, W2, b2):
    src2 = edge_index[0].reshape(NW, EPT)
    dst2 = edge_index[1].reshape(NW, EPT)
    pad_s = jnp.full((NW, EPTP - EPT), N, jnp.int32)
    pad_d = jnp.full((NW, EPTP - EPT), NPAD - 1, jnp.int32)
    src4 = jnp.concatenate([src2, pad_s], axis=1).reshape(NW, NQ, 8, CH)
    dst3 = jnp.concatenate([dst2, pad_d], axis=1).reshape(NW, NCHP, CH)
    zn = jnp.zeros((N,), jnp.float32)
    zr = jnp.zeros((RPT, D), jnp.float32)
    b1r = b1.reshape(1, D)
    b2r = b2.reshape(1, D)

    degp = _deg_kernel(dst2, zn)
    p1, dinv = _tc_first(degp, x, W1)
    a0, a1 = _edge_kernel(src4, dst3, p1, zr)
    p2 = _tc_mid(a0, a1, p1, dinv, b1r, W2)
    c0, c1 = _edge_kernel(src4, dst3, p2, zr)
    out = _tc_last(c0, c1, p2, dinv, b2r)
    return out


# speedup vs baseline: 3.0760x; 2.7557x over previous
import functools

import jax
import jax.numpy as jnp
from jax import lax
from jax.experimental import pallas as pl
from jax.experimental.pallas import tpu as pltpu
from jax.experimental.pallas import tpu_sc as plsc

N = 10000
E = 320000
D = 128
NC = 2
NS = 16
NW = NC * NS
EPT = E // NW
CH = 80
NQ = 16
NCHP = NQ * 8
EPTP = NCHP * CH
NPAD = 10112
RPT = NPAD // NS

_mesh = plsc.VectorSubcoreMesh(
    core_axis_name="c", subcore_axis_name="s", num_cores=NC, num_subcores=NS
)


@functools.partial(
    pl.kernel,
    out_type=jax.ShapeDtypeStruct((NW, N), jnp.float32),
    mesh=_mesh,
    scratch_types=[
        pltpu.VMEM((EPT,), jnp.int32),
        pltpu.VMEM((N,), jnp.float32),
    ],
    compiler_params=pltpu.CompilerParams(needs_layout_passes=False),
)
def _deg_kernel(dst_hbm, zn_hbm, deg_hbm, dstv, degv):
    cid = lax.axis_index("c")
    sid = lax.axis_index("s")
    w = sid * NC + cid
    pltpu.sync_copy(dst_hbm.at[w], dstv)
    pltpu.sync_copy(zn_hbm, degv)
    ones = jnp.ones((16,), jnp.float32)

    @pl.loop(0, EPT // 16)
    def _(i):
        idx = dstv[pl.ds(i * 16, 16)]
        plsc.addupdate_scatter(degv, [idx], ones)

    pltpu.sync_copy(degv, deg_hbm.at[w])


@functools.partial(
    pl.kernel,
    out_type=[
        jax.ShapeDtypeStruct((NPAD, D), jnp.float32),
        jax.ShapeDtypeStruct((NPAD, D), jnp.float32),
    ],
    mesh=_mesh,
    scratch_types=[
        pltpu.VMEM((2, 8, CH), jnp.int32),
        pltpu.VMEM((NCHP, CH), jnp.int32),
        pltpu.VMEM((2 * CH, D), jnp.float32),
        pltpu.VMEM_SHARED((NPAD, D), jnp.float32),
        [pltpu.SemaphoreType.DMA] * 2,
        [pltpu.SemaphoreType.DMA] * 2,
    ],
    compiler_params=pltpu.CompilerParams(needs_layout_passes=False),
)
def _edge_kernel(src_hbm, dst_hbm, p_hbm, zr_hbm, acc0_hbm, acc1_hbm,
                 srcv, dstv, rows, acc_sh, isems, gsems):
    cid = lax.axis_index("c")
    sid = lax.axis_index("s")
    w = sid * NC + cid
    pltpu.sync_copy(dst_hbm.at[w], dstv)
    pltpu.sync_copy(zr_hbm, acc_sh.at[pl.ds(sid * RPT, RPT)])
    plsc.subcore_barrier()

    bufs = [rows.at[pl.ds(b * CH, CH)] for b in range(2)]

    def idx_start(q, s):
        pltpu.async_copy(src_hbm.at[w, q], srcv.at[s], isems[s])

    def idx_wait(q, s):
        pltpu.make_async_copy(src_hbm.at[w, q], srcv.at[s], isems[s]).wait()

    def g_start(s, j):
        pltpu.async_copy(p_hbm.at[srcv.at[s, j]], bufs[j % 2], gsems[j % 2])

    def g_wait(s, j):
        pltpu.make_async_copy(p_hbm.at[srcv.at[s, j]], bufs[j % 2],
                              gsems[j % 2]).wait()

    def scat(c, j):
        pltpu.sync_copy(bufs[j % 2], acc_sh.at[dstv.at[c]], add=True)

    idx_start(0, 0)
    idx_wait(0, 0)
    idx_start(1, 1)
    g_start(0, 0)

    def super_chunk(qbase, s, pf_wait, start_next):
        for j in range(8):
            c = qbase * 8 + j
            if j == 7:
                if pf_wait is not None:
                    idx_wait(pf_wait, s ^ 1)
                    g_start(s ^ 1, 0)
            else:
                g_start(s, j + 1)
            g_wait(s, j)
            scat(c, j)
            if j == 7 and start_next:
                idx_start(pf_wait + 1, s)

    @pl.loop(0, NQ // 2 - 1)
    def _(g):
        q = g * 2
        super_chunk(q, 0, q + 1, True)
        super_chunk(q + 1, 1, q + 2, True)

    q = NQ - 2
    super_chunk(q, 0, q + 1, False)
    super_chunk(q + 1, 1, None, False)

    plsc.subcore_barrier()

    @pl.when(cid == 0)
    def _():
        pltpu.sync_copy(acc_sh.at[pl.ds(sid * RPT, RPT)],
                        acc0_hbm.at[pl.ds(sid * RPT, RPT)])

    @pl.when(cid == 1)
    def _():
        pltpu.sync_copy(acc_sh.at[pl.ds(sid * RPT, RPT)],
                        acc1_hbm.at[pl.ds(sid * RPT, RPT)])


BR = 1024
GRID = (N + BR - 1) // BR


def _row_mask():
    i = pl.program_id(0)
    rid = lax.broadcasted_iota(jnp.int32, (BR, 1), 0) + i * BR
    return rid < N


def _tc_first_body(degp_ref, x_ref, w_ref, p_ref, dinv_ref):
    deg = jnp.sum(degp_ref[...], axis=0) + 1.0
    dinv = lax.rsqrt(deg)[:, None]
    h = jnp.dot(x_ref[...], w_ref[...], preferred_element_type=jnp.float32)
    p_ref[...] = jnp.where(_row_mask(), h * dinv, 0.0)
    dinv_ref[...] = dinv


_tc_first = pl.pallas_call(
    _tc_first_body,
    grid=(GRID,),
    in_specs=[
        pl.BlockSpec((NW, BR), lambda i: (0, i)),
        pl.BlockSpec((BR, D), lambda i: (i, 0)),
        pl.BlockSpec((D, D), lambda i: (0, 0)),
    ],
    out_specs=[
        pl.BlockSpec((BR, D), lambda i: (i, 0)),
        pl.BlockSpec((BR, 1), lambda i: (i, 0)),
    ],
    out_shape=[
        jax.ShapeDtypeStruct((NPAD, D), jnp.float32),
        jax.ShapeDtypeStruct((N, 1), jnp.float32),
    ],
)


def _tc_mid_body(a0_ref, a1_ref, p_ref, dinv_ref, b_ref, w_ref, out_ref):
    dinv = dinv_ref[...]
    z = dinv * (a0_ref[...] + a1_ref[...] + p_ref[...]) + b_ref[...]
    h = jnp.maximum(z, 0.0)
    out_ref[...] = jnp.where(
        _row_mask(),
        jnp.dot(h, w_ref[...], preferred_element_type=jnp.float32) * dinv,
        0.0)


_tc_mid = pl.pallas_call(
    _tc_mid_body,
    grid=(GRID,),
    in_specs=[
        pl.BlockSpec((BR, D), lambda i: (i, 0)),
        pl.BlockSpec((BR, D), lambda i: (i, 0)),
        pl.BlockSpec((BR, D), lambda i: (i, 0)),
        pl.BlockSpec((BR, 1), lambda i: (i, 0)),
        pl.BlockSpec((1, D), lambda i: (0, 0)),
        pl.BlockSpec((D, D), lambda i: (0, 0)),
    ],
    out_specs=pl.BlockSpec((BR, D), lambda i: (i, 0)),
    out_shape=jax.ShapeDtypeStruct((NPAD, D), jnp.float32),
)


def _tc_last_body(a0_ref, a1_ref, p_ref, dinv_ref, b_ref, out_ref):
    z = dinv_ref[...] * (a0_ref[...] + a1_ref[...] + p_ref[...]) + b_ref[...]
    out_ref[...] = z


_tc_last = pl.pallas_call(
    _tc_last_body,
    grid=(GRID,),
    in_specs=[
        pl.BlockSpec((BR, D), lambda i: (i, 0)),
        pl.BlockSpec((BR, D), lambda i: (i, 0)),
        pl.BlockSpec((BR, D), lambda i: (i, 0)),
        pl.BlockSpec((BR, 1), lambda i: (i, 0)),
        pl.BlockSpec((1, D), lambda i: (0, 0)),
    ],
    out_specs=pl.BlockSpec((BR, D), lambda i: (i, 0)),
    out_shape=jax.ShapeDtypeStruct((N, D), jnp.float32),
)


def kernel(x, edge_index, W1, b1, W2, b2):
    src2 = edge_index[0].reshape(NW, EPT)
    dst2 = edge_index[1].reshape(NW, EPT)
    npd = NW * (EPTP - EPT)
    k = jnp.arange(npd, dtype=jnp.int32)
    pad_s = (N + k % (NPAD - N)).reshape(NW, EPTP - EPT)
    pad_d = ((k * 131) % NPAD).reshape(NW, EPTP - EPT)
    src4 = jnp.concatenate([src2, pad_s], axis=1).reshape(NW, NQ, 8, CH)
    dst3 = jnp.concatenate([dst2, pad_d], axis=1).reshape(NW, NCHP, CH)
    zn = jnp.zeros((N,), jnp.float32)
    zr = jnp.zeros((RPT, D), jnp.float32)
    b1r = b1.reshape(1, D)
    b2r = b2.reshape(1, D)

    degp = _deg_kernel(dst2, zn)
    p1, dinv = _tc_first(degp, x, W1)
    a0, a1 = _edge_kernel(src4, dst3, p1, zr)
    p2 = _tc_mid(a0, a1, p1, dinv, b1r, W2)
    c0, c1 = _edge_kernel(src4, dst3, p2, zr)
    out = _tc_last(c0, c1, p2, dinv, b2r)
    return out
